# baseline (speedup 1.0000x reference)
import jax
import jax.numpy as jnp
from jax.experimental import pallas as pl

_B = 256
_POOL = 512
_D = 512
_H = 256
_LMAX = 64
_K = 5
_LCHUNK = 8


def _selector_body(q_ref, keys_ref, w1_ref, b1_ref, g1_ref, be1_ref,
                   w2_ref, b2_ref, wm_ref):
    q = q_ref[...]
    hpre = q * w1_ref[...] + b1_ref[...]
    mean = jnp.mean(hpre, axis=-1, keepdims=True)
    var = jnp.mean((hpre - mean) ** 2, axis=-1, keepdims=True)
    hn = (hpre - mean) / jnp.sqrt(var + 1e-5) * g1_ref[...] + be1_ref[...]
    hact = jnp.maximum(hn, 0.0)
    query = jnp.dot(hact, w2_ref[...],
                    preferred_element_type=jnp.float32) + b2_ref[...]
    qn = query / jnp.maximum(
        jnp.sqrt(jnp.sum(query * query, axis=-1, keepdims=True)), 1e-8)
    keys = keys_ref[...]
    kn = keys / jnp.maximum(
        jnp.sqrt(jnp.sum(keys * keys, axis=-1, keepdims=True)), 1e-8)
    sims = jax.lax.dot_general(qn, kn, (((1,), (1,)), ((), ())),
                               preferred_element_type=jnp.float32)
    scale = 1.0 + 0.5 * jnp.mean(q)
    s = sims * scale
    m = jnp.max(s, axis=1, keepdims=True)
    e = jnp.exp(s - m)
    w = e / jnp.sum(e, axis=1, keepdims=True)

    col = jax.lax.broadcasted_iota(jnp.int32, (_B, _POOL), 1)
    sel = jnp.zeros((_B, _POOL), jnp.bool_)
    for _ in range(_K):
        cur = jnp.where(sel, -jnp.inf, w)
        mk = jnp.max(cur, axis=1, keepdims=True)
        first = jnp.min(jnp.where(cur == mk, col, _POOL),
                        axis=1, keepdims=True)
        sel = jnp.logical_or(sel, col == first)
    wm_ref[...] = jnp.where(sel, w, 0.0)


_W = _LCHUNK * _D


def _combine_body(wm_ref, p_ref, q_ref, o_ref):
    acc = jnp.dot(wm_ref[...], p_ref[...], preferred_element_type=jnp.float32)
    q = q_ref[...]
    length = 5.0 + 59.0 * (1.0 - q / 5.0)
    lengths = jnp.clip(jnp.floor(length).astype(jnp.int32), 5, _LMAX)
    i = pl.program_id(0)
    jcol = jax.lax.broadcasted_iota(jnp.int32, (_B, _W), 1) + i * _W
    o_ref[...] = jnp.where((jcol >> 9) < lengths, acc, 0.0)


def kernel(x_embed, quality_score, prompt_keys, prompt_embeddings,
           W1, b1, g1, be1, W2, b2):
    del x_embed
    w_masked = pl.pallas_call(
        _selector_body,
        out_shape=jax.ShapeDtypeStruct((_B, _POOL), jnp.float32),
    )(quality_score, prompt_keys, W1, b1.reshape(1, _H), g1.reshape(1, _H),
      be1.reshape(1, _H), W2, b2.reshape(1, _D))

    grid = (_LMAX * _D) // _W
    p2d = prompt_embeddings.reshape(_POOL, _LMAX * _D)
    prompted = pl.pallas_call(
        _combine_body,
        grid=(grid,),
        in_specs=[
            pl.BlockSpec((_B, _POOL), lambda i: (0, 0)),
            pl.BlockSpec((_POOL, _W), lambda i: (0, i)),
            pl.BlockSpec((_B, 1), lambda i: (0, 0)),
        ],
        out_specs=pl.BlockSpec((_B, _W), lambda i: (0, i)),
        out_shape=jax.ShapeDtypeStruct((_B, _LMAX * _D), jnp.float32),
    )(w_masked, p2d, quality_score)

    return (prompted.reshape(_B, _LMAX, _D), jnp.zeros((), jnp.float32))


# speedup vs baseline: 2.7881x; 2.7881x over previous
import jax
import jax.numpy as jnp
from jax.experimental import pallas as pl
from jax.experimental.pallas import tpu as pltpu

_B = 256
_POOL = 512
_D = 512
_H = 256
_LMAX = 64
_K = 5
_LCHUNK = 8


def _fused_body(q_ref, keys_ref, w1_ref, b1_ref, g1_ref, be1_ref,
                w2_ref, b2_ref, p_ref, o_ref, wm_ref):
    q = q_ref[...]

    @pl.when(pl.program_id(0) == 0)
    def _selector():
        hpre = q * w1_ref[...] + b1_ref[...]
        mean = jnp.mean(hpre, axis=-1, keepdims=True)
        var = jnp.mean((hpre - mean) ** 2, axis=-1, keepdims=True)
        hn = (hpre - mean) / jnp.sqrt(var + 1e-5) * g1_ref[...] + be1_ref[...]
        hact = jnp.maximum(hn, 0.0)
        query = jnp.dot(hact, w2_ref[...],
                        preferred_element_type=jnp.float32) + b2_ref[...]
        qn = query / jnp.maximum(
            jnp.sqrt(jnp.sum(query * query, axis=-1, keepdims=True)), 1e-8)
        keys = keys_ref[...]
        kn = keys / jnp.maximum(
            jnp.sqrt(jnp.sum(keys * keys, axis=-1, keepdims=True)), 1e-8)
        sims = jax.lax.dot_general(qn, kn, (((1,), (1,)), ((), ())),
                                   preferred_element_type=jnp.float32)
        scale = 1.0 + 0.5 * jnp.mean(q)
        s = sims * scale
        m = jnp.max(s, axis=1, keepdims=True)
        e = jnp.exp(s - m)
        w = e / jnp.sum(e, axis=1, keepdims=True)

        col = jax.lax.broadcasted_iota(jnp.int32, (_B, _POOL), 1)
        sel = jnp.zeros((_B, _POOL), jnp.bool_)
        for _ in range(_K):
            cur = jnp.where(sel, -jnp.inf, w)
            mk = jnp.max(cur, axis=1, keepdims=True)
            first = jnp.min(jnp.where(cur == mk, col, _POOL),
                            axis=1, keepdims=True)
            sel = jnp.logical_or(sel, col == first)
        wm_ref[...] = jnp.where(sel, w, 0.0)

    p = p_ref[...].reshape(_POOL, _LCHUNK * _D)
    acc = jnp.dot(wm_ref[...], p, preferred_element_type=jnp.float32)
    length = 5.0 + 59.0 * (1.0 - q / 5.0)
    lengths = jnp.clip(jnp.floor(length).astype(jnp.int32), 5, _LMAX)
    i = pl.program_id(0)
    lcol = jax.lax.broadcasted_iota(jnp.int32, (_B, _LCHUNK), 1) + i * _LCHUNK
    lm = (lcol < lengths).astype(jnp.float32)
    o_ref[...] = acc.reshape(_B, _LCHUNK, _D) * lm[:, :, None]


def kernel(x_embed, quality_score, prompt_keys, prompt_embeddings,
           W1, b1, g1, be1, W2, b2):
    del x_embed
    const = lambda i: (0, 0)
    prompted = pl.pallas_call(
        _fused_body,
        grid=(_LMAX // _LCHUNK,),
        in_specs=[
            pl.BlockSpec((_B, 1), const),
            pl.BlockSpec((_POOL, _D), const),
            pl.BlockSpec((1, _H), const),
            pl.BlockSpec((1, _H), const),
            pl.BlockSpec((1, _H), const),
            pl.BlockSpec((1, _H), const),
            pl.BlockSpec((_H, _D), const),
            pl.BlockSpec((1, _D), const),
            pl.BlockSpec((_POOL, _LCHUNK, _D), lambda i: (0, i, 0)),
        ],
        out_specs=pl.BlockSpec((_B, _LCHUNK, _D), lambda i: (0, i, 0)),
        out_shape=jax.ShapeDtypeStruct((_B, _LMAX, _D), jnp.float32),
        scratch_shapes=[pltpu.VMEM((_B, _POOL), jnp.float32)],
    )(quality_score, prompt_keys, W1, b1.reshape(1, _H), g1.reshape(1, _H),
      be1.reshape(1, _H), W2, b2.reshape(1, _D), prompt_embeddings)

    return (prompted, jnp.zeros((), jnp.float32))
